# baseline (device time: 82074 ns/iter reference)
import jax
import jax.numpy as jnp
from jax import lax
from jax.experimental import pallas as pl
from jax.experimental.pallas import tpu as pltpu

N_DEV = 8
HPS = 8
DH = 128
SQ = 256
SKV = 4096
D = 1024
SCALE = 0.08838834764831843

GH = 4
N_GROUPS = HPS // GH

_XOR_MASKS = (1, 3, 4)


def kernel(x, Wq, Wo, K_ext, V_ext):
    def body(x_ref, wq_ref, wo_hbm, k_hbm, v_hbm, out_ref,
             k_buf, v_buf, acc_ref, send16, rs_recv, sendag, ag_recv,
             attn_ref, wo_vmem, k_load_sems, v_load_sems, wo_sem,
             rs_send_sems, rs_recv_sems, ag_send_sems, ag_recv_sems):
        my = lax.axis_index("i")
        h0 = my * HPS

        def group_copies(g, slot):
            kc = pltpu.make_async_copy(
                k_hbm.at[0, :, pl.ds(h0 + g * GH, GH), :],
                k_buf.at[slot], k_load_sems.at[slot])
            vc = pltpu.make_async_copy(
                v_hbm.at[0, :, pl.ds(h0 + g * GH, GH), :],
                v_buf.at[slot], v_load_sems.at[slot])
            return kc, vc

        kc0, vc0 = group_copies(0, 0)
        kc0.start()
        vc0.start()
        wo_copy = pltpu.make_async_copy(wo_hbm, wo_vmem, wo_sem)
        wo_copy.start()

        xb = x_ref[0].astype(jnp.bfloat16)
        wqb = wq_ref[...].astype(jnp.bfloat16)
        q_all = jnp.dot(xb, wqb, preferred_element_type=jnp.float32) * SCALE

        for g in range(N_GROUPS):
            slot = g % 2
            if g + 1 < N_GROUPS:
                kcn, vcn = group_copies(g + 1, 1 - slot)
                kcn.start()
                vcn.start()
            kc, vc = group_copies(g, slot)
            kc.wait()
            vc.wait()
            for hh in range(GH):
                h = g * GH + hh
                qh = q_all[:, h * DH:(h + 1) * DH].astype(jnp.bfloat16)
                kh = k_buf[slot, :, hh, :].astype(jnp.bfloat16)
                vh = v_buf[slot, :, hh, :].astype(jnp.bfloat16)
                s = lax.dot_general(qh, kh, (((1,), (1,)), ((), ())),
                                    preferred_element_type=jnp.float32)
                p = jnp.exp(s.astype(jnp.bfloat16))
                li = jnp.sum(p, axis=-1, keepdims=True,
                             dtype=jnp.float32)
                oh = jnp.dot(p, vh,
                             preferred_element_type=jnp.float32) / li
                attn_ref[:, h * DH:(h + 1) * DH] = oh.astype(jnp.bfloat16)

        wo_copy.wait()
        wob = wo_vmem[...].astype(jnp.bfloat16)
        acc_ref[...] = jnp.dot(attn_ref[...], wob,
                               preferred_element_type=jnp.float32)

        SEG = SQ // N_DEV
        own_off = pl.multiple_of(my * SEG, SEG)

        send16[...] = acc_ref[...].astype(jnp.bfloat16)

        for j in range(N_DEV):
            @pl.when(my != j)
            def _():
                rdma = pltpu.make_async_remote_copy(
                    src_ref=send16.at[pl.ds(SEG * j, SEG)],
                    dst_ref=rs_recv.at[my],
                    send_sem=rs_send_sems.at[j],
                    recv_sem=rs_recv_sems.at[my],
                    device_id=(j,),
                    device_id_type=pl.DeviceIdType.MESH,
                )
                rdma.start()

        rs_recv[my] = send16[pl.ds(own_off, SEG), :]

        for j in range(N_DEV):
            @pl.when(my != j)
            def _():
                recv_d = pltpu.make_async_remote_copy(
                    src_ref=send16.at[pl.ds(0, SEG)],
                    dst_ref=rs_recv.at[j],
                    send_sem=rs_send_sems.at[j],
                    recv_sem=rs_recv_sems.at[j],
                    device_id=(j,),
                    device_id_type=pl.DeviceIdType.MESH,
                )
                recv_d.wait_recv()
        reduced = rs_recv[0].astype(jnp.float32)
        for j in range(1, N_DEV):
            reduced = reduced + rs_recv[j].astype(jnp.float32)

        out_ref[0, pl.ds(own_off, SEG), :] = reduced
        sendag[...] = reduced.astype(jnp.bfloat16)

        for j in range(N_DEV):
            @pl.when(my != j)
            def _():
                rdma = pltpu.make_async_remote_copy(
                    src_ref=sendag,
                    dst_ref=ag_recv.at[my],
                    send_sem=ag_send_sems.at[j],
                    recv_sem=ag_recv_sems.at[my],
                    device_id=(j,),
                    device_id_type=pl.DeviceIdType.MESH,
                )
                rdma.start()

        for j in range(N_DEV):
            @pl.when(my != j)
            def _():
                recv_d = pltpu.make_async_remote_copy(
                    src_ref=sendag,
                    dst_ref=ag_recv.at[j],
                    send_sem=ag_send_sems.at[j],
                    recv_sem=ag_recv_sems.at[j],
                    device_id=(j,),
                    device_id_type=pl.DeviceIdType.MESH,
                )
                recv_d.wait_recv()
                out_ref[0, pl.ds(SEG * j, SEG), :] = (
                    ag_recv[j].astype(jnp.float32))

        for j in range(N_DEV):
            @pl.when(my != j)
            def _():
                for sems in (rs_send_sems, ag_send_sems):
                    d = pltpu.make_async_remote_copy(
                        src_ref=sendag,
                        dst_ref=ag_recv.at[j],
                        send_sem=sems.at[j],
                        recv_sem=ag_recv_sems.at[j],
                        device_id=(j,),
                        device_id_type=pl.DeviceIdType.MESH,
                    )
                    d.wait_send()

    return pl.pallas_call(
        body,
        out_shape=jax.ShapeDtypeStruct((1, SQ, D), jnp.float32),
        in_specs=[
            pl.BlockSpec(memory_space=pltpu.VMEM),
            pl.BlockSpec(memory_space=pltpu.VMEM),
            pl.BlockSpec(memory_space=pl.ANY),
            pl.BlockSpec(memory_space=pl.ANY),
            pl.BlockSpec(memory_space=pl.ANY),
        ],
        out_specs=pl.BlockSpec(memory_space=pltpu.VMEM),
        scratch_shapes=[
            pltpu.VMEM((2, SKV, GH, DH), jnp.float32),
            pltpu.VMEM((2, SKV, GH, DH), jnp.float32),
            pltpu.VMEM((SQ, D), jnp.float32),
            pltpu.VMEM((SQ, D), jnp.bfloat16),
            pltpu.VMEM((N_DEV, SQ // N_DEV, D), jnp.bfloat16),
            pltpu.VMEM((SQ // N_DEV, D), jnp.bfloat16),
            pltpu.VMEM((N_DEV, SQ // N_DEV, D), jnp.bfloat16),
            pltpu.VMEM((SQ, D), jnp.bfloat16),
            pltpu.VMEM((D, D), jnp.float32),
            pltpu.SemaphoreType.DMA((2,)),
            pltpu.SemaphoreType.DMA((2,)),
            pltpu.SemaphoreType.DMA(()),
            pltpu.SemaphoreType.DMA((N_DEV,)),
            pltpu.SemaphoreType.DMA((N_DEV,)),
            pltpu.SemaphoreType.DMA((N_DEV,)),
            pltpu.SemaphoreType.DMA((N_DEV,)),
        ],
        compiler_params=pltpu.CompilerParams(
            vmem_limit_bytes=100 * 1024 * 1024,
        ),
    )(x, Wq, Wo, K_ext, V_ext)


# device time: 78612 ns/iter; 1.0440x vs baseline; 1.0440x over previous
import jax
import jax.numpy as jnp
from jax import lax
from jax.experimental import pallas as pl
from jax.experimental.pallas import tpu as pltpu

N_DEV = 8
HPS = 8
DH = 128
SQ = 256
SKV = 4096
D = 1024
SCALE = 0.08838834764831843

GH = 4
N_GROUPS = HPS // GH

_XOR_MASKS = (1, 3, 4)


def kernel(x, Wq, Wo, K_ext, V_ext):
    def body(x_ref, wq_ref, wo_ref, k_hbm, v_hbm, out_ref,
             k_buf, v_buf, acc_ref, send16, rs_recv, sendag, ag_recv,
             attn_ref, k_load_sems, v_load_sems,
             rs_send_sems, rs_recv_sems, ag_send_sems, ag_recv_sems):
        my = lax.axis_index("i")
        h0 = my * HPS

        def group_copies(g, slot):
            kc = pltpu.make_async_copy(
                k_hbm.at[0, :, pl.ds(h0 + g * GH, GH), :],
                k_buf.at[slot], k_load_sems.at[slot])
            vc = pltpu.make_async_copy(
                v_hbm.at[0, :, pl.ds(h0 + g * GH, GH), :],
                v_buf.at[slot], v_load_sems.at[slot])
            return kc, vc

        kc0, vc0 = group_copies(0, 0)
        kc0.start()
        vc0.start()

        xb = x_ref[0].astype(jnp.bfloat16)
        wqb = wq_ref[...].astype(jnp.bfloat16)
        q_all = jnp.dot(xb, wqb, preferred_element_type=jnp.float32) * SCALE

        for g in range(N_GROUPS):
            slot = g % 2
            if g + 1 < N_GROUPS:
                kcn, vcn = group_copies(g + 1, 1 - slot)
                kcn.start()
                vcn.start()
            kc, vc = group_copies(g, slot)
            kc.wait()
            vc.wait()
            for hh in range(GH):
                h = g * GH + hh
                qh = q_all[:, h * DH:(h + 1) * DH].astype(jnp.bfloat16)
                kh = k_buf[slot, :, hh, :].astype(jnp.bfloat16)
                vh = v_buf[slot, :, hh, :].astype(jnp.bfloat16)
                s = lax.dot_general(qh, kh, (((1,), (1,)), ((), ())),
                                    preferred_element_type=jnp.float32)
                p = jnp.exp(s)
                li = jnp.sum(p, axis=-1, keepdims=True)
                oh = jnp.dot(p.astype(jnp.bfloat16), vh,
                             preferred_element_type=jnp.float32) / li
                attn_ref[:, h * DH:(h + 1) * DH] = oh.astype(jnp.bfloat16)

        wob = wo_ref[...].astype(jnp.bfloat16)
        acc_ref[...] = jnp.dot(attn_ref[...], wob,
                               preferred_element_type=jnp.float32)

        SEG = SQ // N_DEV
        own_off = pl.multiple_of(my * SEG, SEG)

        send16[...] = acc_ref[...].astype(jnp.bfloat16)

        for j in range(N_DEV):
            @pl.when(my != j)
            def _():
                rdma = pltpu.make_async_remote_copy(
                    src_ref=send16.at[pl.ds(SEG * j, SEG)],
                    dst_ref=rs_recv.at[my],
                    send_sem=rs_send_sems.at[j],
                    recv_sem=rs_recv_sems.at[my],
                    device_id=(j,),
                    device_id_type=pl.DeviceIdType.MESH,
                )
                rdma.start()

        rs_recv[my] = send16[pl.ds(own_off, SEG), :]

        for j in range(N_DEV):
            @pl.when(my != j)
            def _():
                recv_d = pltpu.make_async_remote_copy(
                    src_ref=send16.at[pl.ds(0, SEG)],
                    dst_ref=rs_recv.at[j],
                    send_sem=rs_send_sems.at[j],
                    recv_sem=rs_recv_sems.at[j],
                    device_id=(j,),
                    device_id_type=pl.DeviceIdType.MESH,
                )
                recv_d.wait_recv()
        reduced = rs_recv[0].astype(jnp.float32)
        for j in range(1, N_DEV):
            reduced = reduced + rs_recv[j].astype(jnp.float32)

        out_ref[0, pl.ds(own_off, SEG), :] = reduced
        sendag[...] = reduced.astype(jnp.bfloat16)

        for j in range(N_DEV):
            @pl.when(my != j)
            def _():
                rdma = pltpu.make_async_remote_copy(
                    src_ref=sendag,
                    dst_ref=ag_recv.at[my],
                    send_sem=ag_send_sems.at[j],
                    recv_sem=ag_recv_sems.at[my],
                    device_id=(j,),
                    device_id_type=pl.DeviceIdType.MESH,
                )
                rdma.start()

        for j in range(N_DEV):
            @pl.when(my != j)
            def _():
                recv_d = pltpu.make_async_remote_copy(
                    src_ref=sendag,
                    dst_ref=ag_recv.at[j],
                    send_sem=ag_send_sems.at[j],
                    recv_sem=ag_recv_sems.at[j],
                    device_id=(j,),
                    device_id_type=pl.DeviceIdType.MESH,
                )
                recv_d.wait_recv()
                out_ref[0, pl.ds(SEG * j, SEG), :] = (
                    ag_recv[j].astype(jnp.float32))

        for j in range(N_DEV):
            @pl.when(my != j)
            def _():
                for sems in (rs_send_sems, ag_send_sems):
                    d = pltpu.make_async_remote_copy(
                        src_ref=sendag,
                        dst_ref=ag_recv.at[j],
                        send_sem=sems.at[j],
                        recv_sem=ag_recv_sems.at[j],
                        device_id=(j,),
                        device_id_type=pl.DeviceIdType.MESH,
                    )
                    d.wait_send()

    return pl.pallas_call(
        body,
        out_shape=jax.ShapeDtypeStruct((1, SQ, D), jnp.float32),
        in_specs=[
            pl.BlockSpec(memory_space=pltpu.VMEM),
            pl.BlockSpec(memory_space=pltpu.VMEM),
            pl.BlockSpec(memory_space=pltpu.VMEM),
            pl.BlockSpec(memory_space=pl.ANY),
            pl.BlockSpec(memory_space=pl.ANY),
        ],
        out_specs=pl.BlockSpec(memory_space=pltpu.VMEM),
        scratch_shapes=[
            pltpu.VMEM((2, SKV, GH, DH), jnp.float32),
            pltpu.VMEM((2, SKV, GH, DH), jnp.float32),
            pltpu.VMEM((SQ, D), jnp.float32),
            pltpu.VMEM((SQ, D), jnp.bfloat16),
            pltpu.VMEM((N_DEV, SQ // N_DEV, D), jnp.bfloat16),
            pltpu.VMEM((SQ // N_DEV, D), jnp.bfloat16),
            pltpu.VMEM((N_DEV, SQ // N_DEV, D), jnp.bfloat16),
            pltpu.VMEM((SQ, D), jnp.bfloat16),
            pltpu.SemaphoreType.DMA((2,)),
            pltpu.SemaphoreType.DMA((2,)),
            pltpu.SemaphoreType.DMA((N_DEV,)),
            pltpu.SemaphoreType.DMA((N_DEV,)),
            pltpu.SemaphoreType.DMA((N_DEV,)),
            pltpu.SemaphoreType.DMA((N_DEV,)),
        ],
        compiler_params=pltpu.CompilerParams(
            vmem_limit_bytes=100 * 1024 * 1024,
        ),
    )(x, Wq, Wo, K_ext, V_ext)
